# Initial kernel scaffold; baseline (speedup 1.0000x reference)
import functools

import jax
import jax.numpy as jnp
from jax import lax
from jax.experimental import pallas as pl
from jax.experimental.pallas import tpu as pltpu
from jax.experimental.pallas import tpu_sc as plsc

N_SP = 1048576
B = 16384
U = 512
C = 8
D = 64
NW = 32
BT = B // NW
SUB = 128
NSUB = BT // SUB
MEMT = N_SP // NW

_info = plsc.get_sparse_core_info()
_NC = _info.num_cores


def _sc_body(mem_in, lt_in, xf_in, candf_in, idx_in,
             mem_out, mpart_out, npart_out, noisep_out,
             m_acc, n_acc, xbuf, ltbuf, qbuf, candbuf, idxbuf, valbuf,
             membuf, nbuf):
    wid = lax.axis_index("s") * _NC + lax.axis_index("c")
    lanes = lax.iota(jnp.int32, 16)
    zero16 = jnp.zeros((16,), jnp.float32)

    def _zm(i, _):
        for j in range(4):
            m_acc[i, pl.ds(16 * j, 16)] = zero16
        return 0
    lax.fori_loop(0, U, _zm, 0)

    def _zn(i, _):
        n_acc[pl.ds(i * 16, 16)] = zero16
        return 0
    lax.fori_loop(0, U // 16, _zn, 0)

    nvec = zero16
    for s in range(NSUB):
        base = wid * BT + s * SUB
        pltpu.sync_copy(xf_in.at[pl.ds(base * D, SUB * D)], xbuf)
        pltpu.sync_copy(lt_in.at[:, pl.ds(base, SUB)], ltbuf)
        pltpu.sync_copy(candf_in.at[pl.ds(base * C, SUB * C)], candbuf)

        def _smax(g, acc):
            col = pl.ds(g * 16, 16)
            ls = [ltbuf[c, col] for c in range(C + 1)]
            mx = ls[0]
            for c in range(1, C + 1):
                mx = jnp.maximum(mx, ls[c])
            es = [jnp.exp(l - mx) for l in ls]
            tot = es[0]
            for c in range(1, C + 1):
                tot = tot + es[c]
            r = 1.0 / tot
            rowbase = (lanes + g * 16) * C
            for c in range(C):
                plsc.store_scatter(qbuf, [rowbase + c], es[c] * r)
            return acc + es[C] * r

        nvec = lax.fori_loop(0, SUB // 16, _smax, nvec)

        def _nacc(v, _):
            cv = candbuf[pl.ds(v * 16, 16)]
            qv = qbuf[pl.ds(v * 16, 16)]
            plsc.addupdate_scatter(n_acc, [cv], qv)
            return 0
        lax.fori_loop(0, SUB * C // 16, _nacc, 0)

        def _mrow(rr, _):
            xb = rr * D
            xv = [xbuf[pl.ds(xb + 16 * j, 16)] for j in range(4)]
            for c in range(C):
                q = qbuf[rr * C + c]
                u = candbuf[rr * C + c]
                urow = jnp.full((16,), u, jnp.int32)
                for j in range(4):
                    plsc.addupdate_scatter(
                        m_acc, [urow, lanes + 16 * j], q * xv[j])
            return 0
        lax.fori_loop(0, SUB, _mrow, 0)

    pltpu.sync_copy(n_acc, npart_out.at[wid])
    pltpu.sync_copy(m_acc, mpart_out.at[wid])
    nbuf[...] = zero16 + jnp.sum(nvec)
    pltpu.sync_copy(nbuf, noisep_out.at[wid])

    lo = wid * MEMT
    pltpu.sync_copy(mem_in.at[pl.ds(lo, MEMT)], membuf)
    pltpu.sync_copy(idx_in, idxbuf)
    pltpu.sync_copy(lt_in.at[C], valbuf)

    def _scan(v, _):
        iv = idxbuf[pl.ds(v * 16, 16)]
        vals = valbuf[pl.ds(v * 16, 16)]
        mask = (iv >= lo) & (iv < lo + MEMT)
        local = jnp.where(mask, iv - lo, 0)
        plsc.store_scatter(membuf, [local], vals, mask=mask)
        return 0
    lax.fori_loop(0, B // 16, _scan, 0)

    pltpu.sync_copy(membuf, mem_out.at[pl.ds(lo, MEMT)])


_sc_call = functools.partial(
    pl.kernel,
    out_type=[
        jax.ShapeDtypeStruct((N_SP,), jnp.float32),
        jax.ShapeDtypeStruct((NW, U, D), jnp.float32),
        jax.ShapeDtypeStruct((NW, U), jnp.float32),
        jax.ShapeDtypeStruct((NW, 16), jnp.float32),
    ],
    mesh=plsc.VectorSubcoreMesh(core_axis_name="c", subcore_axis_name="s"),
    scratch_types=[
        pltpu.VMEM((U, D), jnp.float32),
        pltpu.VMEM((U,), jnp.float32),
        pltpu.VMEM((SUB * D,), jnp.float32),
        pltpu.VMEM((C + 1, SUB), jnp.float32),
        pltpu.VMEM((SUB * C,), jnp.float32),
        pltpu.VMEM((SUB * C,), jnp.int32),
        pltpu.VMEM((B,), jnp.int32),
        pltpu.VMEM((B,), jnp.float32),
        pltpu.VMEM((MEMT,), jnp.float32),
        pltpu.VMEM((16,), jnp.float32),
    ],
)(_sc_body)


def _tc_body(mp_ref, np_ref, noi_ref, m_out, n_out, s_out):
    npart = np_ref[...]
    n = jnp.sum(npart, axis=0)
    n_out[...] = n[None, :]
    mm = jnp.sum(mp_ref[...], axis=0)
    m_out[...] = mm / jnp.maximum(n, 1.0)[:, None]
    s_out[...] = jnp.full((1, 1), jnp.sum(noi_ref[...]), jnp.float32)


_tc_call = pl.pallas_call(
    _tc_body,
    out_shape=[
        jax.ShapeDtypeStruct((U, D), jnp.float32),
        jax.ShapeDtypeStruct((1, U), jnp.float32),
        jax.ShapeDtypeStruct((1, 1), jnp.float32),
    ],
)


def kernel(mem, x, logits, idx, candidates):
    xf = x.reshape(-1).astype(jnp.float32)
    lt = logits.T.astype(jnp.float32)
    candf = candidates.astype(jnp.int32).reshape(-1)
    idx32 = idx.astype(jnp.int32)
    mem_new, mpart, npart, noisep = _sc_call(
        mem.astype(jnp.float32), lt, xf, candf, idx32)
    m2, n2, s2 = _tc_call(mpart, npart, noisep)
    return (mem_new, n2.reshape(U), m2.reshape(U, 2, 32),
            s2.reshape(()))


# speedup vs baseline: 52.5734x; 52.5734x over previous
import functools

import jax
import jax.numpy as jnp
from jax import lax
from jax.experimental import pallas as pl
from jax.experimental.pallas import tpu as pltpu
from jax.experimental.pallas import tpu_sc as plsc

N_SP = 1048576
B = 16384
U = 512
C = 8
D = 64
NW = 32
BT = B // NW
SUB = 128
NSUB = BT // SUB
MEMT = N_SP // NW

_info = plsc.get_sparse_core_info()
_NC = _info.num_cores


def _sc_body(mem_in, lt_in, xf_in, candf_in, idx_in,
             mem_out, mpart_out, npart_out, noisep_out,
             m_acc, n_acc, xbuf, ltbuf, qbuf, candbuf, idxbuf, valbuf,
             membuf, nbuf):
    wid = lax.axis_index("s") * _NC + lax.axis_index("c")
    lanes = lax.iota(jnp.int32, 16)
    zero16 = jnp.zeros((16,), jnp.float32)

    def _zm(i, _):
        m_acc[pl.ds(i * 16, 16)] = zero16
        return 0
    lax.fori_loop(0, U * D // 16, _zm, 0)

    def _zn(i, _):
        n_acc[pl.ds(i * 16, 16)] = zero16
        return 0
    lax.fori_loop(0, U // 16, _zn, 0)

    nvec = zero16
    for s in range(NSUB):
        base = wid * BT + s * SUB
        pltpu.sync_copy(xf_in.at[pl.ds(base * D, SUB * D)], xbuf)
        for c in range(C + 1):
            pltpu.sync_copy(lt_in.at[pl.ds(c * B + base, SUB)],
                            ltbuf.at[pl.ds(c * SUB, SUB)])
        pltpu.sync_copy(candf_in.at[pl.ds(base * C, SUB * C)], candbuf)

        def _smax(g, acc):
            ls = [ltbuf[pl.ds(c * SUB + g * 16, 16)] for c in range(C + 1)]
            mx = ls[0]
            for c in range(1, C + 1):
                mx = jnp.maximum(mx, ls[c])
            es = [jnp.exp(l - mx) for l in ls]
            tot = es[0]
            for c in range(1, C + 1):
                tot = tot + es[c]
            r = 1.0 / tot
            rowbase = (lanes + g * 16) * C
            for c in range(C):
                plsc.store_scatter(qbuf, [rowbase + c], es[c] * r)
            return acc + es[C] * r

        nvec = lax.fori_loop(0, SUB // 16, _smax, nvec)

        def _nacc(v, _):
            cv = candbuf[pl.ds(v * 16, 16)]
            qv = qbuf[pl.ds(v * 16, 16)]
            plsc.addupdate_scatter(n_acc, [cv], qv)
            return 0
        lax.fori_loop(0, SUB * C // 16, _nacc, 0)

        def _mrow(t, _):
            qv = qbuf[pl.ds(t * 16, 16)]
            cv = candbuf[pl.ds(t * 16, 16)]
            for half in range(2):
                rr = 2 * t + half
                xb = rr * D
                xv = [xbuf[pl.ds(xb + 16 * j, 16)] for j in range(4)]
                for c in range(C):
                    q = qv[C * half + c]
                    u = cv[C * half + c]
                    ubase = jnp.full((16,), u * D, jnp.int32)
                    for j in range(4):
                        plsc.addupdate_scatter(
                            m_acc, [ubase + (lanes + 16 * j)], q * xv[j])
            return 0
        lax.fori_loop(0, SUB // 2, _mrow, 0)

    pltpu.sync_copy(n_acc, npart_out.at[pl.ds(wid * U, U)])
    pltpu.sync_copy(m_acc, mpart_out.at[pl.ds(wid * U * D, U * D)])
    nbuf[...] = zero16 + jnp.sum(nvec)
    pltpu.sync_copy(nbuf, noisep_out.at[pl.ds(wid * 16, 16)])

    lo = wid * MEMT
    pltpu.sync_copy(mem_in.at[pl.ds(lo, MEMT)], membuf)
    pltpu.sync_copy(idx_in, idxbuf)
    pltpu.sync_copy(lt_in.at[pl.ds(C * B, B)], valbuf)

    def _scan(v, _):
        iv = idxbuf[pl.ds(v * 16, 16)]
        vals = valbuf[pl.ds(v * 16, 16)]
        mask = (iv >= lo) & (iv < lo + MEMT)
        local = jnp.where(mask, iv - lo, 0)
        plsc.store_scatter(membuf, [local], vals, mask=mask)
        return 0
    lax.fori_loop(0, B // 16, _scan, 0)

    pltpu.sync_copy(membuf, mem_out.at[pl.ds(lo, MEMT)])


_sc_call = functools.partial(
    pl.kernel,
    out_type=[
        jax.ShapeDtypeStruct((N_SP,), jnp.float32),
        jax.ShapeDtypeStruct((NW * U * D,), jnp.float32),
        jax.ShapeDtypeStruct((NW * U,), jnp.float32),
        jax.ShapeDtypeStruct((NW * 16,), jnp.float32),
    ],
    mesh=plsc.VectorSubcoreMesh(core_axis_name="c", subcore_axis_name="s"),
    compiler_params=pltpu.CompilerParams(needs_layout_passes=False),
    scratch_types=[
        pltpu.VMEM((U * D,), jnp.float32),
        pltpu.VMEM((U,), jnp.float32),
        pltpu.VMEM((SUB * D,), jnp.float32),
        pltpu.VMEM(((C + 1) * SUB,), jnp.float32),
        pltpu.VMEM((SUB * C,), jnp.float32),
        pltpu.VMEM((SUB * C,), jnp.int32),
        pltpu.VMEM((B,), jnp.int32),
        pltpu.VMEM((B,), jnp.float32),
        pltpu.VMEM((MEMT,), jnp.float32),
        pltpu.VMEM((16,), jnp.float32),
    ],
)(_sc_body)


def _tc_body(mp_ref, np_ref, noi_ref, m_out, n_out, s_out):
    npart = np_ref[...]
    n = jnp.sum(npart, axis=0)
    n_out[...] = n[None, :]
    mm = jnp.sum(mp_ref[...], axis=0)
    m_out[...] = mm / jnp.maximum(n, 1.0)[:, None]
    s_out[...] = jnp.full((1, 1), jnp.sum(noi_ref[...]) * (1.0 / 16.0),
                          jnp.float32)


_tc_call = pl.pallas_call(
    _tc_body,
    out_shape=[
        jax.ShapeDtypeStruct((U, D), jnp.float32),
        jax.ShapeDtypeStruct((1, U), jnp.float32),
        jax.ShapeDtypeStruct((1, 1), jnp.float32),
    ],
)


def kernel(mem, x, logits, idx, candidates):
    xf = x.reshape(-1).astype(jnp.float32)
    lt = logits.T.reshape(-1).astype(jnp.float32)
    candf = candidates.astype(jnp.int32).reshape(-1)
    idx32 = idx.astype(jnp.int32)
    mem_new, mpart, npart, noisep = _sc_call(
        mem.astype(jnp.float32), lt, xf, candf, idx32)
    m2, n2, s2 = _tc_call(mpart.reshape(NW, U, D), npart.reshape(NW, U),
                          noisep.reshape(NW, 16))
    return (mem_new, n2.reshape(U), m2.reshape(U, 2, 32),
            s2.reshape(()))


# speedup vs baseline: 85.0399x; 1.6175x over previous
import functools

import jax
import jax.numpy as jnp
from jax import lax
from jax.experimental import pallas as pl
from jax.experimental.pallas import tpu as pltpu
from jax.experimental.pallas import tpu_sc as plsc

N_SP = 1048576
B = 16384
U = 512
C = 8
D = 64
NW = 32
BT = B // NW
CH = 32
NCH = BT // CH
SEG = 2048
NSEG = B // SEG
MEMT = N_SP // NW

_info = plsc.get_sparse_core_info()
_NC = _info.num_cores


def _sc_body(mem_in, x_in, lg_in, nls_in, cand_in, idx_in,
             mem_out, mpart_out, npart_out, noisep_out,
             m_acc, n_acc, xb0, xb1, lb0, lb1, cb0, cb1, qbuf,
             ib0, ib1, vb0, vb1, membuf, nbuf,
             semA0, semA1, semB0, semB1, semM, semP):
    wid = lax.axis_index("s") * _NC + lax.axis_index("c")
    lanes = lax.iota(jnp.int32, 16)
    zero16 = jnp.zeros((16,), jnp.float32)
    zero16i = jnp.zeros((16,), jnp.int32)
    lo = wid * MEMT
    xb = (xb0, xb1)
    lb = (lb0, lb1)
    cb = (cb0, cb1)
    ib = (ib0, ib1)
    vb = (vb0, vb1)
    semA = (semA0, semA1)
    semB = (semB0, semB1)

    def issue_a(c, b):
        base = wid * BT + c * CH
        pltpu.async_copy(x_in.at[pl.ds(base, CH)], xb[b], semA[b])
        pltpu.async_copy(lg_in.at[pl.ds(base, CH)], lb[b], semA[b])
        pltpu.async_copy(cand_in.at[pl.ds(base, CH)], cb[b], semA[b])

    def wait_a(b):
        pltpu.make_async_copy(x_in.at[pl.ds(0, CH)], xb[b], semA[b]).wait()
        pltpu.make_async_copy(lg_in.at[pl.ds(0, CH)], lb[b], semA[b]).wait()
        pltpu.make_async_copy(cand_in.at[pl.ds(0, CH)], cb[b], semA[b]).wait()

    def issue_b(s, b):
        pltpu.async_copy(idx_in.at[pl.ds(s * SEG, SEG)], ib[b], semB[b])
        pltpu.async_copy(nls_in.at[pl.ds(s * SEG, SEG)], vb[b], semB[b])

    def wait_b(b):
        pltpu.make_async_copy(idx_in.at[pl.ds(0, SEG)], ib[b], semB[b]).wait()
        pltpu.make_async_copy(nls_in.at[pl.ds(0, SEG)], vb[b], semB[b]).wait()

    issue_a(0, 0)
    issue_a(1, 1)
    pltpu.async_copy(mem_in.at[pl.ds(lo, MEMT)], membuf, semM)
    issue_b(0, 0)
    issue_b(1, 1)

    def _zm(i, _):
        for j in range(4):
            m_acc[pl.ds(i * 64 + 16 * j, 16)] = zero16
        return 0
    lax.fori_loop(0, U * D // 64, _zm, 0)

    def _zn(i, _):
        n_acc[0, pl.ds(i * 16, 16)] = zero16
        return 0
    lax.fori_loop(0, U // 16, _zn, 0)

    rowinc = (lanes >= 8).astype(jnp.int32)
    colsel = lanes & 7

    def _chunk(i, acc):
        for b in range(2):
            c = 2 * i + b
            wait_a(b)
            xr, lr, cr = xb[b], lb[b], cb[b]

            for g in range(CH // 16):
                rows = lanes + g * 16
                ls = [plsc.load_gather(lr, [rows, zero16i + cc])
                      for cc in range(C + 1)]
                mx = ls[0]
                for cc in range(1, C + 1):
                    mx = jnp.maximum(mx, ls[cc])
                es = [jnp.exp(l - mx) for l in ls]
                tot = es[0]
                for cc in range(1, C + 1):
                    tot = tot + es[cc]
                r = 1.0 / tot
                rowbase = rows * C
                for cc in range(C):
                    plsc.store_scatter(qbuf, [rowbase + cc], es[cc] * r)
                acc = acc + es[C] * r

            def _nacc(v, _):
                rvec = rowinc + 2 * v
                cv = plsc.load_gather(cr, [rvec, colsel])
                qv = qbuf[pl.ds(v * 16, 16)]
                plsc.addupdate_scatter(n_acc, [zero16i, cv], qv)
                return 0
            lax.fori_loop(0, CH * C // 16, _nacc, 0)

            def _mrow(t, _):
                rvec = rowinc + 2 * t
                cv = plsc.load_gather(cr, [rvec, colsel])
                qv = qbuf[pl.ds(t * 16, 16)]
                for half in range(2):
                    rr = 2 * t + half
                    xv = [xr[rr, jr, pl.ds(16 * jc, 16)]
                          for jr in range(2) for jc in range(2)]
                    for cc in range(C):
                        q = qv[C * half + cc]
                        u = cv[C * half + cc]
                        ubase = jnp.full((16,), u * D, jnp.int32)
                        for j in range(4):
                            plsc.addupdate_scatter(
                                m_acc, [ubase + (lanes + 16 * j)], q * xv[j])
                return 0
            lax.fori_loop(0, CH // 2, _mrow, 0)

            @pl.when(c + 2 < NCH)
            def _():
                issue_a(c + 2, b)
        return acc

    nvec = lax.fori_loop(0, NCH // 2, _chunk, zero16)

    pltpu.async_copy(n_acc, npart_out.at[pl.ds(wid, 1)], semP)
    pltpu.async_copy(m_acc, mpart_out.at[pl.ds(wid * U * D, U * D)], semP)
    nbuf[0, pl.ds(0, 16)] = zero16 + jnp.sum(nvec)
    pltpu.async_copy(nbuf, noisep_out.at[pl.ds(wid, 1)], semP)

    pltpu.make_async_copy(mem_in.at[pl.ds(lo, MEMT)], membuf, semM).wait()

    def _segs(i, _):
        for b in range(2):
            s = 2 * i + b
            wait_b(b)
            ir, vr = ib[b], vb[b]

            def _scan(v, _):
                iv = ir[pl.ds(v * 16, 16)]
                vals = vr[pl.ds(v * 16, 16)]
                mask = (iv >= lo) & (iv < lo + MEMT)
                local = jnp.where(mask, iv - lo, 0)
                plsc.store_scatter(membuf, [local], vals, mask=mask)
                return 0
            lax.fori_loop(0, SEG // 16, _scan, 0)

            @pl.when(s + 2 < NSEG)
            def _():
                issue_b(s + 2, b)
        return 0

    lax.fori_loop(0, NSEG // 2, _segs, 0)

    pltpu.sync_copy(membuf, mem_out.at[pl.ds(lo, MEMT)])
    pltpu.make_async_copy(n_acc, npart_out.at[pl.ds(0, 1)], semP).wait()
    pltpu.make_async_copy(
        m_acc, mpart_out.at[pl.ds(0, U * D)], semP).wait()
    pltpu.make_async_copy(nbuf, noisep_out.at[pl.ds(0, 1)], semP).wait()


_sc_call = functools.partial(
    pl.kernel,
    out_type=[
        jax.ShapeDtypeStruct((N_SP,), jnp.float32),
        jax.ShapeDtypeStruct((NW * U * D,), jnp.float32),
        jax.ShapeDtypeStruct((NW, U), jnp.float32),
        jax.ShapeDtypeStruct((NW, 16), jnp.float32),
    ],
    mesh=plsc.VectorSubcoreMesh(core_axis_name="c", subcore_axis_name="s"),
    compiler_params=pltpu.CompilerParams(needs_layout_passes=False),
    scratch_types=[
        pltpu.VMEM((U * D,), jnp.float32),
        pltpu.VMEM((1, U), jnp.float32),
        pltpu.VMEM((CH, 2, 32), jnp.float32),
        pltpu.VMEM((CH, 2, 32), jnp.float32),
        pltpu.VMEM((CH, C + 1), jnp.float32),
        pltpu.VMEM((CH, C + 1), jnp.float32),
        pltpu.VMEM((CH, C), jnp.int32),
        pltpu.VMEM((CH, C), jnp.int32),
        pltpu.VMEM((CH * C,), jnp.float32),
        pltpu.VMEM((SEG,), jnp.int32),
        pltpu.VMEM((SEG,), jnp.int32),
        pltpu.VMEM((SEG,), jnp.float32),
        pltpu.VMEM((SEG,), jnp.float32),
        pltpu.VMEM((MEMT,), jnp.float32),
        pltpu.VMEM((1, 16), jnp.float32),
        pltpu.SemaphoreType.DMA,
        pltpu.SemaphoreType.DMA,
        pltpu.SemaphoreType.DMA,
        pltpu.SemaphoreType.DMA,
        pltpu.SemaphoreType.DMA,
        pltpu.SemaphoreType.DMA,
    ],
)(_sc_body)


def _tc_body(mp_ref, np_ref, noi_ref, m_out, n_out, s_out):
    i = pl.program_id(0)

    @pl.when(i == 0)
    def _():
        m_out[...] = mp_ref[...]
        n = jnp.sum(np_ref[...], axis=0)
        n_out[...] = n[None, :]
        s_out[...] = jnp.full((1, 1), jnp.sum(noi_ref[...]) * (1.0 / 16.0),
                              jnp.float32)

    @pl.when(i > 0)
    def _():
        m_out[...] += mp_ref[...]


_tc_call = pl.pallas_call(
    _tc_body,
    grid=(NW,),
    in_specs=[
        pl.BlockSpec((U * D,), lambda i: (i,)),
        pl.BlockSpec((NW, U), lambda i: (0, 0)),
        pl.BlockSpec((NW, 16), lambda i: (0, 0)),
    ],
    out_specs=[
        pl.BlockSpec((U * D,), lambda i: (0,)),
        pl.BlockSpec((1, U), lambda i: (0, 0)),
        pl.BlockSpec((1, 1), lambda i: (0, 0)),
    ],
    out_shape=[
        jax.ShapeDtypeStruct((U * D,), jnp.float32),
        jax.ShapeDtypeStruct((1, U), jnp.float32),
        jax.ShapeDtypeStruct((1, 1), jnp.float32),
    ],
)


def kernel(mem, x, logits, idx, candidates):
    nls = logits[:, C]
    cand = candidates.astype(jnp.int32)
    idx32 = idx.astype(jnp.int32)
    mem_new, mpart, npart, noisep = _sc_call(
        mem.astype(jnp.float32), x, logits, nls, cand, idx32)
    msum, n2, s2 = _tc_call(mpart, npart, noisep)
    n = n2.reshape(U)
    m = (msum.reshape(U, D) / jnp.maximum(n, 1.0)[:, None]).reshape(U, 2, 32)
    return mem_new, n, m, s2.reshape(())


# speedup vs baseline: 86.7388x; 1.0200x over previous
import functools

import jax
import jax.numpy as jnp
from jax import lax
from jax.experimental import pallas as pl
from jax.experimental.pallas import tpu as pltpu
from jax.experimental.pallas import tpu_sc as plsc

N_SP = 1048576
B = 16384
U = 512
C = 8
D = 64
NW = 32
BT = B // NW
CH = 16
NCH = BT // CH
SEG = 1024
NSEG = B // SEG
MEMT = N_SP // NW

_info = plsc.get_sparse_core_info()
_NC = _info.num_cores


def _sc_body(mem_in, x_in, lg_in, nls_in, cand_in, idx_in, zeros_in,
             mem_out, mpart_out, npart_out, noisep_out,
             m_acc, n_acc, xb0, xb1, lb0, lb1, cb0, cb1, qbuf,
             ib0, ib1, vb0, vb1, membuf, nbuf,
             semA0, semA1, semB0, semB1, semM, semZ, semP):
    wid = lax.axis_index("s") * _NC + lax.axis_index("c")
    lanes = lax.iota(jnp.int32, 16)
    zero16 = jnp.zeros((16,), jnp.float32)
    zero16i = jnp.zeros((16,), jnp.int32)
    lo = wid * MEMT
    xb = (xb0, xb1)
    lb = (lb0, lb1)
    cb = (cb0, cb1)
    ib = (ib0, ib1)
    vb = (vb0, vb1)
    semA = (semA0, semA1)
    semB = (semB0, semB1)

    def issue_a(c, b):
        base = wid * BT + c * CH
        pltpu.async_copy(x_in.at[pl.ds(base, CH)], xb[b], semA[b])
        pltpu.async_copy(lg_in.at[pl.ds(base, CH)], lb[b], semA[b])
        pltpu.async_copy(cand_in.at[pl.ds(base, CH)], cb[b], semA[b])

    def wait_a(b):
        pltpu.make_async_copy(x_in.at[pl.ds(0, CH)], xb[b], semA[b]).wait()
        pltpu.make_async_copy(lg_in.at[pl.ds(0, CH)], lb[b], semA[b]).wait()
        pltpu.make_async_copy(cand_in.at[pl.ds(0, CH)], cb[b], semA[b]).wait()

    def issue_b(s, b):
        pltpu.async_copy(idx_in.at[pl.ds(s * SEG, SEG)], ib[b], semB[b])
        pltpu.async_copy(nls_in.at[pl.ds(s * SEG, SEG)], vb[b], semB[b])

    def wait_b(b):
        pltpu.make_async_copy(idx_in.at[pl.ds(0, SEG)], ib[b], semB[b]).wait()
        pltpu.make_async_copy(nls_in.at[pl.ds(0, SEG)], vb[b], semB[b]).wait()

    issue_a(0, 0)
    issue_a(1, 1)
    pltpu.async_copy(zeros_in, m_acc, semZ)
    pltpu.async_copy(mem_in.at[pl.ds(lo, MEMT)], membuf, semM)
    issue_b(0, 0)
    issue_b(1, 1)

    def _zn(i, _):
        n_acc[0, pl.ds(i * 16, 16)] = zero16
        return 0
    lax.fori_loop(0, U // 16, _zn, 0)

    pltpu.make_async_copy(zeros_in, m_acc, semZ).wait()

    rowinc = (lanes >= 8).astype(jnp.int32)
    colsel = lanes & 7

    def _chunk(i, acc):
        for b in range(2):
            c = 2 * i + b
            wait_a(b)
            xr, lr, cr = xb[b], lb[b], cb[b]

            for g in range(CH // 16):
                rows = lanes + g * 16
                ls = [plsc.load_gather(lr, [rows, zero16i + cc])
                      for cc in range(C + 1)]
                mx = ls[0]
                for cc in range(1, C + 1):
                    mx = jnp.maximum(mx, ls[cc])
                es = [jnp.exp(l - mx) for l in ls]
                tot = es[0]
                for cc in range(1, C + 1):
                    tot = tot + es[cc]
                r = 1.0 / tot
                rowbase = rows * C
                for cc in range(C):
                    plsc.store_scatter(qbuf, [rowbase + cc], es[cc] * r)
                acc = acc + es[C] * r

            def _mrow(t, _):
                rvec = rowinc + 2 * t
                cv = plsc.load_gather(cr, [rvec, colsel])
                qv = qbuf[pl.ds(t * 16, 16)]
                plsc.addupdate_scatter(n_acc, [zero16i, cv], qv)
                for half in range(2):
                    rr = 2 * t + half
                    xv = [xr[rr, jr, pl.ds(16 * jc, 16)]
                          for jr in range(2) for jc in range(2)]
                    for cc in range(C):
                        q = qv[C * half + cc]
                        u = cv[C * half + cc]
                        ubase = jnp.full((16,), u * D, jnp.int32)
                        for j in range(4):
                            plsc.addupdate_scatter(
                                m_acc, [ubase + (lanes + 16 * j)], q * xv[j])
                return 0
            lax.fori_loop(0, CH // 2, _mrow, 0)

            @pl.when(c + 2 < NCH)
            def _():
                issue_a(c + 2, b)
        return acc

    nvec = lax.fori_loop(0, NCH // 2, _chunk, zero16)

    pltpu.async_copy(n_acc, npart_out.at[pl.ds(wid, 1)], semP)
    pltpu.async_copy(m_acc, mpart_out.at[pl.ds(wid * U * D, U * D)], semP)
    nbuf[0, pl.ds(0, 16)] = zero16 + jnp.sum(nvec)
    pltpu.async_copy(nbuf, noisep_out.at[pl.ds(wid, 1)], semP)

    pltpu.make_async_copy(mem_in.at[pl.ds(lo, MEMT)], membuf, semM).wait()

    def _segs(i, _):
        for b in range(2):
            s = 2 * i + b
            wait_b(b)
            ir, vr = ib[b], vb[b]

            def _scan(v, _):
                iv = ir[pl.ds(v * 16, 16)]
                vals = vr[pl.ds(v * 16, 16)]
                mask = (iv >= lo) & (iv < lo + MEMT)
                local = jnp.where(mask, iv - lo, 0)
                plsc.store_scatter(membuf, [local], vals, mask=mask)
                return 0
            lax.fori_loop(0, SEG // 16, _scan, 0)

            @pl.when(s + 2 < NSEG)
            def _():
                issue_b(s + 2, b)
        return 0

    lax.fori_loop(0, NSEG // 2, _segs, 0)

    pltpu.sync_copy(membuf, mem_out.at[pl.ds(lo, MEMT)])
    pltpu.make_async_copy(n_acc, npart_out.at[pl.ds(0, 1)], semP).wait()
    pltpu.make_async_copy(
        m_acc, mpart_out.at[pl.ds(0, U * D)], semP).wait()
    pltpu.make_async_copy(nbuf, noisep_out.at[pl.ds(0, 1)], semP).wait()


_sc_call = functools.partial(
    pl.kernel,
    out_type=[
        jax.ShapeDtypeStruct((N_SP,), jnp.float32),
        jax.ShapeDtypeStruct((NW * U * D,), jnp.float32),
        jax.ShapeDtypeStruct((NW, U), jnp.float32),
        jax.ShapeDtypeStruct((NW, 16), jnp.float32),
    ],
    mesh=plsc.VectorSubcoreMesh(core_axis_name="c", subcore_axis_name="s"),
    compiler_params=pltpu.CompilerParams(
        needs_layout_passes=False, use_tc_tiling_on_sc=True),
    scratch_types=[
        pltpu.VMEM((U * D,), jnp.float32),
        pltpu.VMEM((1, U), jnp.float32),
        pltpu.VMEM((CH, 2, 32), jnp.float32),
        pltpu.VMEM((CH, 2, 32), jnp.float32),
        pltpu.VMEM((CH, C + 1), jnp.float32),
        pltpu.VMEM((CH, C + 1), jnp.float32),
        pltpu.VMEM((CH, C), jnp.int32),
        pltpu.VMEM((CH, C), jnp.int32),
        pltpu.VMEM((CH * C,), jnp.float32),
        pltpu.VMEM((SEG,), jnp.int32),
        pltpu.VMEM((SEG,), jnp.int32),
        pltpu.VMEM((SEG,), jnp.float32),
        pltpu.VMEM((SEG,), jnp.float32),
        pltpu.VMEM((MEMT,), jnp.float32),
        pltpu.VMEM((1, 16), jnp.float32),
        pltpu.SemaphoreType.DMA,
        pltpu.SemaphoreType.DMA,
        pltpu.SemaphoreType.DMA,
        pltpu.SemaphoreType.DMA,
        pltpu.SemaphoreType.DMA,
        pltpu.SemaphoreType.DMA,
        pltpu.SemaphoreType.DMA,
    ],
)(_sc_body)


def _tc_body(mp_ref, np_ref, noi_ref, m_out, n_out, s_out):
    acc = mp_ref[pl.ds(0, U * D)]
    for k in range(1, NW):
        acc = acc + mp_ref[pl.ds(k * U * D, U * D)]
    m_out[...] = acc
    n = jnp.sum(np_ref[...], axis=0)
    n_out[...] = n[None, :]
    s_out[...] = jnp.full((1, 1), jnp.sum(noi_ref[...]) * (1.0 / 16.0),
                          jnp.float32)


_tc_call = pl.pallas_call(
    _tc_body,
    out_shape=[
        jax.ShapeDtypeStruct((U * D,), jnp.float32),
        jax.ShapeDtypeStruct((1, U), jnp.float32),
        jax.ShapeDtypeStruct((1, 1), jnp.float32),
    ],
)


def kernel(mem, x, logits, idx, candidates):
    nls = logits[:, C]
    cand = candidates.astype(jnp.int32)
    idx32 = idx.astype(jnp.int32)
    zeros = jnp.zeros((U * D,), jnp.float32)
    mem_new, mpart, npart, noisep = _sc_call(
        mem.astype(jnp.float32), x, logits, nls, cand, idx32, zeros)
    msum, n2, s2 = _tc_call(mpart, npart, noisep)
    n = n2.reshape(U)
    m = (msum.reshape(U, D) / jnp.maximum(n, 1.0)[:, None]).reshape(U, 2, 32)
    return mem_new, n, m, s2.reshape(())


# speedup vs baseline: 91.4185x; 1.0540x over previous
import functools

import jax
import jax.numpy as jnp
from jax import lax
from jax.experimental import pallas as pl
from jax.experimental.pallas import tpu as pltpu
from jax.experimental.pallas import tpu_sc as plsc

N_SP = 1048576
B = 16384
U = 512
C = 8
D = 64
NW = 32
BT = B // NW
CH = 32
NCH = BT // CH
MEMT = N_SP // NW

_info = plsc.get_sparse_core_info()
_NC = _info.num_cores
_MESH = plsc.VectorSubcoreMesh(core_axis_name="c", subcore_axis_name="s")
_PARAMS = pltpu.CompilerParams(needs_layout_passes=False)


def _sc_a_body(x_in, lg_in, cand_in, zeros_in,
               mpart_out, npart_out, noisep_out,
               m_acc, n_acc, xb0, xb1, lb0, lb1, cb0, cb1, qbuf, nbuf,
               semA0, semA1, semZ, semP):
    wid = lax.axis_index("s") * _NC + lax.axis_index("c")
    lanes = lax.iota(jnp.int32, 16)
    zero16 = jnp.zeros((16,), jnp.float32)
    zero16i = jnp.zeros((16,), jnp.int32)
    xb = (xb0, xb1)
    lb = (lb0, lb1)
    cb = (cb0, cb1)
    semA = (semA0, semA1)

    def issue_a(c, b):
        base = wid * BT + c * CH
        pltpu.async_copy(x_in.at[pl.ds(base, CH)], xb[b], semA[b])
        pltpu.async_copy(lg_in.at[pl.ds(base, CH)], lb[b], semA[b])
        pltpu.async_copy(cand_in.at[pl.ds(base, CH)], cb[b], semA[b])

    def wait_a(b):
        pltpu.make_async_copy(x_in.at[pl.ds(0, CH)], xb[b], semA[b]).wait()
        pltpu.make_async_copy(lg_in.at[pl.ds(0, CH)], lb[b], semA[b]).wait()
        pltpu.make_async_copy(cand_in.at[pl.ds(0, CH)], cb[b], semA[b]).wait()

    issue_a(0, 0)
    issue_a(1, 1)
    pltpu.async_copy(zeros_in, m_acc, semZ)

    def _zn(i, _):
        n_acc[0, pl.ds(i * 16, 16)] = zero16
        return 0
    lax.fori_loop(0, U // 16, _zn, 0)

    pltpu.make_async_copy(zeros_in, m_acc, semZ).wait()

    rowinc = (lanes >= 8).astype(jnp.int32)
    colsel = lanes & 7

    def _chunk(i, acc):
        for b in range(2):
            c = 2 * i + b
            wait_a(b)
            xr, lr, cr = xb[b], lb[b], cb[b]

            def _smax(g, a):
                rows = lanes + g * 16
                ls = [plsc.load_gather(lr, [rows, zero16i + cc])
                      for cc in range(C + 1)]
                mx = ls[0]
                for cc in range(1, C + 1):
                    mx = jnp.maximum(mx, ls[cc])
                es = [jnp.exp(l - mx) for l in ls]
                tot = es[0]
                for cc in range(1, C + 1):
                    tot = tot + es[cc]
                r = 1.0 / tot
                rowbase = rows * C
                for cc in range(C):
                    plsc.store_scatter(qbuf, [rowbase + cc], es[cc] * r)
                return a + es[C] * r

            acc = lax.fori_loop(0, CH // 16, _smax, acc)

            def _mrow(t, _):
                rvec = rowinc + 2 * t
                cv = plsc.load_gather(cr, [rvec, colsel])
                qv = qbuf[pl.ds(t * 16, 16)]
                plsc.addupdate_scatter(n_acc, [zero16i, cv], qv)
                for half in range(2):
                    rr = 2 * t + half
                    xv = [xr[rr, jr, pl.ds(16 * jc, 16)]
                          for jr in range(2) for jc in range(2)]
                    for cc in range(C):
                        q = qv[C * half + cc]
                        u = cv[C * half + cc]
                        ubase = jnp.full((16,), u * D, jnp.int32)
                        for j in range(4):
                            plsc.addupdate_scatter(
                                m_acc, [ubase + (lanes + 16 * j)], q * xv[j])
                return 0
            lax.fori_loop(0, CH // 2, _mrow, 0)

            @pl.when(c + 2 < NCH)
            def _():
                issue_a(c + 2, b)
        return acc

    nvec = lax.fori_loop(0, NCH // 2, _chunk, zero16)

    pltpu.async_copy(n_acc, npart_out.at[pl.ds(wid, 1)], semP)
    pltpu.async_copy(m_acc, mpart_out.at[pl.ds(wid * U * D, U * D)], semP)
    nbuf[0, pl.ds(0, 16)] = zero16 + jnp.sum(nvec)
    pltpu.async_copy(nbuf, noisep_out.at[pl.ds(wid, 1)], semP)
    pltpu.make_async_copy(n_acc, npart_out.at[pl.ds(0, 1)], semP).wait()
    pltpu.make_async_copy(m_acc, mpart_out.at[pl.ds(0, U * D)], semP).wait()
    pltpu.make_async_copy(nbuf, noisep_out.at[pl.ds(0, 1)], semP).wait()


_sc_a_call = functools.partial(
    pl.kernel,
    out_type=[
        jax.ShapeDtypeStruct((NW * U * D,), jnp.float32),
        jax.ShapeDtypeStruct((NW, U), jnp.float32),
        jax.ShapeDtypeStruct((NW, 16), jnp.float32),
    ],
    mesh=_MESH,
    compiler_params=_PARAMS,
    scratch_types=[
        pltpu.VMEM((U * D,), jnp.float32),
        pltpu.VMEM((1, U), jnp.float32),
        pltpu.VMEM((CH, 2, 32), jnp.float32),
        pltpu.VMEM((CH, 2, 32), jnp.float32),
        pltpu.VMEM((CH, C + 1), jnp.float32),
        pltpu.VMEM((CH, C + 1), jnp.float32),
        pltpu.VMEM((CH, C), jnp.int32),
        pltpu.VMEM((CH, C), jnp.int32),
        pltpu.VMEM((CH * C,), jnp.float32),
        pltpu.VMEM((1, 16), jnp.float32),
        pltpu.SemaphoreType.DMA,
        pltpu.SemaphoreType.DMA,
        pltpu.SemaphoreType.DMA,
        pltpu.SemaphoreType.DMA,
    ],
)(_sc_a_body)


SEG = 2048
NSEG = B // SEG


def _sc_b_body(mem_in, nls_in, idx_in, mem_out,
               ib0, ib1, vb0, vb1, membuf, semM, semB0, semB1):
    wid = lax.axis_index("s") * _NC + lax.axis_index("c")
    lo = wid * MEMT
    ib = (ib0, ib1)
    vb = (vb0, vb1)
    semB = (semB0, semB1)

    def issue_b(s, b):
        pltpu.async_copy(idx_in.at[pl.ds(s * SEG, SEG)], ib[b], semB[b])
        pltpu.async_copy(nls_in.at[pl.ds(s * SEG, SEG)], vb[b], semB[b])

    def wait_b(b):
        pltpu.make_async_copy(idx_in.at[pl.ds(0, SEG)], ib[b], semB[b]).wait()
        pltpu.make_async_copy(nls_in.at[pl.ds(0, SEG)], vb[b], semB[b]).wait()

    pltpu.async_copy(mem_in.at[pl.ds(lo, MEMT)], membuf, semM)
    issue_b(0, 0)
    issue_b(1, 1)
    pltpu.make_async_copy(mem_in.at[pl.ds(lo, MEMT)], membuf, semM).wait()

    def _segs(i, _):
        for b in range(2):
            s = 2 * i + b
            wait_b(b)
            ir, vr = ib[b], vb[b]

            def _scan(v, _):
                iv = ir[pl.ds(v * 16, 16)]
                vals = vr[pl.ds(v * 16, 16)]
                mask = (iv >= lo) & (iv < lo + MEMT)
                local = jnp.where(mask, iv - lo, 0)
                plsc.store_scatter(membuf, [local], vals, mask=mask)
                return 0
            lax.fori_loop(0, SEG // 16, _scan, 0)

            @pl.when(s + 2 < NSEG)
            def _():
                issue_b(s + 2, b)
        return 0

    lax.fori_loop(0, NSEG // 2, _segs, 0)

    pltpu.sync_copy(membuf, mem_out.at[pl.ds(lo, MEMT)])


_sc_b_call = functools.partial(
    pl.kernel,
    out_type=[jax.ShapeDtypeStruct((N_SP,), jnp.float32)],
    mesh=_MESH,
    compiler_params=_PARAMS,
    scratch_types=[
        pltpu.VMEM((SEG,), jnp.int32),
        pltpu.VMEM((SEG,), jnp.int32),
        pltpu.VMEM((SEG,), jnp.float32),
        pltpu.VMEM((SEG,), jnp.float32),
        pltpu.VMEM((MEMT,), jnp.float32),
        pltpu.SemaphoreType.DMA,
        pltpu.SemaphoreType.DMA,
        pltpu.SemaphoreType.DMA,
    ],
)(_sc_b_body)


def _tc_body(mp_ref, np_ref, noi_ref, m_out, n_out, s_out):
    acc = mp_ref[pl.ds(0, U * D)]
    for k in range(1, NW):
        acc = acc + mp_ref[pl.ds(k * U * D, U * D)]
    m_out[...] = acc
    n = jnp.sum(np_ref[...], axis=0)
    n_out[...] = n[None, :]
    s_out[...] = jnp.full((1, 1), jnp.sum(noi_ref[...]) * (1.0 / 16.0),
                          jnp.float32)


_tc_call = pl.pallas_call(
    _tc_body,
    out_shape=[
        jax.ShapeDtypeStruct((U * D,), jnp.float32),
        jax.ShapeDtypeStruct((1, U), jnp.float32),
        jax.ShapeDtypeStruct((1, 1), jnp.float32),
    ],
)


def kernel(mem, x, logits, idx, candidates):
    nls = logits[:, C]
    cand = candidates.astype(jnp.int32)
    idx32 = idx.astype(jnp.int32)
    zeros = jnp.zeros((U * D,), jnp.float32)
    (mem_new,) = _sc_b_call(mem.astype(jnp.float32), nls, idx32)
    mpart, npart, noisep = _sc_a_call(x, logits, cand, zeros)
    msum, n2, s2 = _tc_call(mpart, npart, noisep)
    n = n2.reshape(U)
    m = (msum.reshape(U, D) / jnp.maximum(n, 1.0)[:, None]).reshape(U, 2, 32)
    return mem_new, n, m, s2.reshape(())


# speedup vs baseline: 104.9971x; 1.1485x over previous
import functools

import jax
import jax.numpy as jnp
from jax import lax
from jax.experimental import pallas as pl
from jax.experimental.pallas import tpu as pltpu
from jax.experimental.pallas import tpu_sc as plsc

N_SP = 1048576
B = 16384
U = 512
C = 8
D = 64
NW = 32
BT = B // NW
CH = 32
NCH = BT // CH
MEMT = N_SP // NW

_info = plsc.get_sparse_core_info()
_NC = _info.num_cores
_MESH = plsc.VectorSubcoreMesh(core_axis_name="c", subcore_axis_name="s")
_PARAMS = pltpu.CompilerParams(needs_layout_passes=False)


def _sc_a_body(x_in, lg_in, cand_in, zeros_in, tok_in,
               mpart_out, npart_out, noisep_out,
               m_acc, n_acc, xb0, xb1, lb0, lb1, cb0, cb1, qbuf, nbuf,
               semA0, semA1, semZ, semP):
    wid = lax.axis_index("s") * _NC + lax.axis_index("c")
    lanes = lax.iota(jnp.int32, 16)
    zero16 = jnp.zeros((16,), jnp.float32)
    zero16i = jnp.zeros((16,), jnp.int32)
    xb = (xb0, xb1)
    lb = (lb0, lb1)
    cb = (cb0, cb1)
    semA = (semA0, semA1)

    def issue_a(c, b):
        base = wid * BT + c * CH
        pltpu.async_copy(x_in.at[pl.ds(base, CH)], xb[b], semA[b])
        pltpu.async_copy(lg_in.at[pl.ds(base, CH)], lb[b], semA[b])
        pltpu.async_copy(cand_in.at[pl.ds(base, CH)], cb[b], semA[b])

    def wait_a(b):
        pltpu.make_async_copy(x_in.at[pl.ds(0, CH)], xb[b], semA[b]).wait()
        pltpu.make_async_copy(lg_in.at[pl.ds(0, CH)], lb[b], semA[b]).wait()
        pltpu.make_async_copy(cand_in.at[pl.ds(0, CH)], cb[b], semA[b]).wait()

    issue_a(0, 0)
    issue_a(1, 1)
    pltpu.async_copy(zeros_in, m_acc, semZ)

    def _zn(i, _):
        n_acc[0, pl.ds(i * 16, 16)] = zero16
        return 0
    lax.fori_loop(0, U // 16, _zn, 0)

    pltpu.make_async_copy(zeros_in, m_acc, semZ).wait()

    rowinc = (lanes >= 8).astype(jnp.int32)
    colsel = lanes & 7

    def _chunk(i, acc):
        for b in range(2):
            c = 2 * i + b
            wait_a(b)
            xr, lr, cr = xb[b], lb[b], cb[b]

            def _smax(g, a):
                rows = lanes + g * 16
                ls = [plsc.load_gather(lr, [rows, zero16i + cc])
                      for cc in range(C + 1)]
                mx = ls[0]
                for cc in range(1, C + 1):
                    mx = jnp.maximum(mx, ls[cc])
                es = [jnp.exp(l - mx) for l in ls]
                tot = es[0]
                for cc in range(1, C + 1):
                    tot = tot + es[cc]
                r = 1.0 / tot
                rowbase = rows * C
                for cc in range(C):
                    plsc.store_scatter(qbuf, [rowbase + cc], es[cc] * r)
                return a + es[C] * r

            acc = lax.fori_loop(0, CH // 16, _smax, acc)

            def _mrow(t, _):
                rvec = rowinc + 2 * t
                cv = plsc.load_gather(cr, [rvec, colsel])
                qv = qbuf[pl.ds(t * 16, 16)]
                plsc.addupdate_scatter(n_acc, [zero16i, cv], qv)
                for half in range(2):
                    rr = 2 * t + half
                    xv = [xr[rr, jr, pl.ds(16 * jc, 16)]
                          for jr in range(2) for jc in range(2)]
                    for cc in range(C):
                        q = qv[C * half + cc]
                        u = cv[C * half + cc]
                        ubase = jnp.full((16,), u * D, jnp.int32)
                        for j in range(4):
                            plsc.addupdate_scatter(
                                m_acc, [ubase + (lanes + 16 * j)], q * xv[j])
                return 0
            lax.fori_loop(0, CH // 2, _mrow, 0)

            @pl.when(c + 2 < NCH)
            def _():
                issue_a(c + 2, b)
        return acc

    nvec = lax.fori_loop(0, NCH // 2, _chunk, zero16)

    pltpu.async_copy(n_acc, npart_out.at[pl.ds(wid, 1)], semP)
    pltpu.async_copy(m_acc, mpart_out.at[pl.ds(wid * U * D, U * D)], semP)
    nbuf[0, pl.ds(0, 16)] = zero16 + jnp.sum(nvec)
    pltpu.async_copy(nbuf, noisep_out.at[pl.ds(wid, 1)], semP)
    pltpu.make_async_copy(n_acc, npart_out.at[pl.ds(0, 1)], semP).wait()
    pltpu.make_async_copy(m_acc, mpart_out.at[pl.ds(0, U * D)], semP).wait()
    pltpu.make_async_copy(nbuf, noisep_out.at[pl.ds(0, 1)], semP).wait()


_sc_a_call = functools.partial(
    pl.kernel,
    out_type=[
        jax.ShapeDtypeStruct((NW * U * D,), jnp.float32),
        jax.ShapeDtypeStruct((NW, U), jnp.float32),
        jax.ShapeDtypeStruct((NW, 16), jnp.float32),
    ],
    mesh=_MESH,
    compiler_params=_PARAMS,
    scratch_types=[
        pltpu.VMEM((U * D,), jnp.float32),
        pltpu.VMEM((1, U), jnp.float32),
        pltpu.VMEM((CH, 2, 32), jnp.float32),
        pltpu.VMEM((CH, 2, 32), jnp.float32),
        pltpu.VMEM((CH, C + 1), jnp.float32),
        pltpu.VMEM((CH, C + 1), jnp.float32),
        pltpu.VMEM((CH, C), jnp.int32),
        pltpu.VMEM((CH, C), jnp.int32),
        pltpu.VMEM((CH * C,), jnp.float32),
        pltpu.VMEM((1, 16), jnp.float32),
        pltpu.SemaphoreType.DMA,
        pltpu.SemaphoreType.DMA,
        pltpu.SemaphoreType.DMA,
        pltpu.SemaphoreType.DMA,
    ],
)(_sc_a_body)


SEG = 2048
NSEG = B // SEG


def _sc_b_body(mem_in, nls_in, idx_in, mem_out, tok_out,
               ib0, ib1, vb0, vb1, membuf, semM, semB0, semB1):
    wid = lax.axis_index("s") * _NC + lax.axis_index("c")
    lo = wid * MEMT
    ib = (ib0, ib1)
    vb = (vb0, vb1)
    semB = (semB0, semB1)

    def issue_b(s, b):
        pltpu.async_copy(idx_in.at[pl.ds(s * SEG, SEG)], ib[b], semB[b])
        pltpu.async_copy(nls_in.at[pl.ds(s * SEG, SEG)], vb[b], semB[b])

    def wait_b(b):
        pltpu.make_async_copy(idx_in.at[pl.ds(0, SEG)], ib[b], semB[b]).wait()
        pltpu.make_async_copy(nls_in.at[pl.ds(0, SEG)], vb[b], semB[b]).wait()

    pltpu.async_copy(mem_in.at[pl.ds(lo, MEMT)], membuf, semM)
    issue_b(0, 0)
    issue_b(1, 1)
    pltpu.make_async_copy(mem_in.at[pl.ds(lo, MEMT)], membuf, semM).wait()

    def _segs(i, _):
        for b in range(2):
            s = 2 * i + b
            wait_b(b)
            ir, vr = ib[b], vb[b]

            def _scan(v, _):
                iv = ir[pl.ds(v * 16, 16)]
                vals = vr[pl.ds(v * 16, 16)]
                mask = (iv >= lo) & (iv < lo + MEMT)
                local = jnp.where(mask, iv - lo, 0)
                plsc.store_scatter(membuf, [local], vals, mask=mask)
                return 0
            lax.fori_loop(0, SEG // 16, _scan, 0)

            @pl.when(s + 2 < NSEG)
            def _():
                issue_b(s + 2, b)
        return 0

    lax.fori_loop(0, NSEG // 2, _segs, 0)

    pltpu.sync_copy(membuf, mem_out.at[pl.ds(lo, MEMT)])

    @pl.when(wid == 0)
    def _():
        pltpu.sync_copy(membuf.at[pl.ds(0, 16)], tok_out)


_sc_b_call = functools.partial(
    pl.kernel,
    out_type=[jax.ShapeDtypeStruct((N_SP,), jnp.float32),
              jax.ShapeDtypeStruct((16,), jnp.float32)],
    mesh=_MESH,
    compiler_params=_PARAMS,
    scratch_types=[
        pltpu.VMEM((SEG,), jnp.int32),
        pltpu.VMEM((SEG,), jnp.int32),
        pltpu.VMEM((SEG,), jnp.float32),
        pltpu.VMEM((SEG,), jnp.float32),
        pltpu.VMEM((MEMT,), jnp.float32),
        pltpu.SemaphoreType.DMA,
        pltpu.SemaphoreType.DMA,
        pltpu.SemaphoreType.DMA,
    ],
)(_sc_b_body)


def _tc_body(mp_ref, np_ref, noi_ref, m_out, n_out, s_out):
    acc = mp_ref[pl.ds(0, U * D)]
    for k in range(1, NW):
        acc = acc + mp_ref[pl.ds(k * U * D, U * D)]
    m_out[...] = acc
    n = jnp.sum(np_ref[...], axis=0)
    n_out[...] = n[None, :]
    s_out[...] = jnp.full((1, 1), jnp.sum(noi_ref[...]) * (1.0 / 16.0),
                          jnp.float32)


_tc_call = pl.pallas_call(
    _tc_body,
    out_shape=[
        jax.ShapeDtypeStruct((U * D,), jnp.float32),
        jax.ShapeDtypeStruct((1, U), jnp.float32),
        jax.ShapeDtypeStruct((1, 1), jnp.float32),
    ],
)


def kernel(mem, x, logits, idx, candidates):
    nls = logits[:, C]
    cand = candidates.astype(jnp.int32)
    idx32 = idx.astype(jnp.int32)
    zeros = jnp.zeros((U * D,), jnp.float32)
    mem_new, tok = _sc_b_call(mem.astype(jnp.float32), nls, idx32)
    mpart, npart, noisep = _sc_a_call(x, logits, cand, zeros, tok)
    msum, n2, s2 = _tc_call(mpart, npart, noisep)
    n = n2.reshape(U)
    m = (msum.reshape(U, D) / jnp.maximum(n, 1.0)[:, None]).reshape(U, 2, 32)
    return mem_new, n, m, s2.reshape(())
